# Initial kernel scaffold; baseline (speedup 1.0000x reference)
import jax
import jax.numpy as jnp
from jax.experimental import pallas as pl


def kernel(boxes, scores):
    raise NotImplementedError("write your pallas kernel here")


# speedup vs baseline: 8.0840x; 8.0840x over previous
import jax
import jax.numpy as jnp
from jax.experimental import pallas as pl
from jax.experimental.pallas import tpu as pltpu

_SCORE_THRESH = 0.05
_NMS_THRESH = 0.5
_DET_PER_IMG = 300
_TOPK = 1000
_PAD = 1024


def _nms_kernel(bc_ref, br_ref, sc_ref, out_ref, sup_ref, keep_ref):
    x1c = bc_ref[:, 0:1]
    y1c = bc_ref[:, 1:2]
    x2c = bc_ref[:, 2:3]
    y2c = bc_ref[:, 3:4]
    x1r = br_ref[0:1, :]
    y1r = br_ref[1:2, :]
    x2r = br_ref[2:3, :]
    y2r = br_ref[3:4, :]
    area_c = (x2c - x1c) * (y2c - y1c)
    area_r = (x2r - x1r) * (y2r - y1r)
    iw = jnp.maximum(jnp.minimum(x2c, x2r) - jnp.maximum(x1c, x1r), 0.0)
    ih = jnp.maximum(jnp.minimum(y2c, y2r) - jnp.maximum(y1c, y1r), 0.0)
    inter = iw * ih
    iou = inter / (area_c + area_r - inter + 1e-8)
    sup_ref[:, :] = jnp.where(iou > _NMS_THRESH, 1.0, 0.0)
    keep_ref[:, :] = jnp.ones((1, _PAD), jnp.float32)
    lane = jax.lax.broadcasted_iota(jnp.int32, (1, _PAD), 1)

    def body(i, carry):
        row = sup_ref[pl.ds(i, 1), :]
        keep = keep_ref[:, :]
        onehot = jnp.where(lane == i, 1.0, 0.0)
        ki = jnp.sum(keep * onehot)
        later = jnp.where(lane > i, 1.0, 0.0)
        keep_ref[:, :] = keep * (1.0 - row * later * ki)
        return carry

    jax.lax.fori_loop(0, _TOPK, body, 0)

    keep = keep_ref[:, :]
    sc = sc_ref[:, :]
    valid = (keep > 0.5) & (sc > _SCORE_THRESH)
    out_ref[:, :] = jnp.where(valid, sc, -jnp.inf)


def _run_nms(bc, br, sc):
    return pl.pallas_call(
        _nms_kernel,
        out_shape=jax.ShapeDtypeStruct((1, _PAD), jnp.float32),
        scratch_shapes=[
            pltpu.VMEM((_PAD, _PAD), jnp.float32),
            pltpu.VMEM((1, _PAD), jnp.float32),
        ],
    )(bc, br, sc)


def kernel(boxes, scores):
    masked = jnp.where(scores > _SCORE_THRESH, scores, -jnp.inf)
    top_scores, top_idx = jax.lax.top_k(masked, _TOPK)
    tb = boxes[top_idx]
    x1 = tb[:, 0] * 800.0
    y1 = tb[:, 1] * 800.0
    x2 = x1 + tb[:, 2] * 512.0 + 1.0
    y2 = y1 + tb[:, 3] * 512.0 + 1.0
    bxyxy = jnp.stack([x1, y1, x2, y2], axis=1)

    bc = jnp.pad(bxyxy, ((0, _PAD - _TOPK), (0, 0)))
    br = bc.T
    sc = jnp.pad(top_scores, (0, _PAD - _TOPK), constant_values=-jnp.inf)[None, :]

    final = _run_nms(bc, br, sc)[0, :_TOPK]
    det_scores, det_idx = jax.lax.top_k(final, _DET_PER_IMG)
    det_boxes = bxyxy[det_idx]
    det_scores = jnp.where(jnp.isfinite(det_scores), det_scores, 0.0)
    return jnp.concatenate([det_boxes, det_scores[:, None]], axis=1)


# speedup vs baseline: 8.1246x; 1.0050x over previous
import jax
import jax.numpy as jnp
from jax.experimental import pallas as pl
from jax.experimental.pallas import tpu as pltpu

_SCORE_THRESH = 0.05
_NMS_THRESH = 0.5
_DET_PER_IMG = 300
_TOPK = 1000
_PAD = 1024


def _nms_kernel(bc_ref, br_ref, sc_ref, out_ref, sup_ref, keep_ref):
    x1c = bc_ref[:, 0:1]
    y1c = bc_ref[:, 1:2]
    x2c = bc_ref[:, 2:3]
    y2c = bc_ref[:, 3:4]
    x1r = br_ref[0:1, :]
    y1r = br_ref[1:2, :]
    x2r = br_ref[2:3, :]
    y2r = br_ref[3:4, :]
    area_c = (x2c - x1c) * (y2c - y1c)
    area_r = (x2r - x1r) * (y2r - y1r)
    iw = jnp.maximum(jnp.minimum(x2c, x2r) - jnp.maximum(x1c, x1r), 0.0)
    ih = jnp.maximum(jnp.minimum(y2c, y2r) - jnp.maximum(y1c, y1r), 0.0)
    inter = iw * ih
    iou = inter / (area_c + area_r - inter + 1e-8)
    row_iota = jax.lax.broadcasted_iota(jnp.int32, (_PAD, _PAD), 0)
    col_iota = jax.lax.broadcasted_iota(jnp.int32, (_PAD, _PAD), 1)
    sup_ref[:, :] = jnp.where(
        (iou > _NMS_THRESH) & (col_iota > row_iota), 1.0, 0.0
    )
    keep_ref[:, :] = jnp.ones((1, _PAD), jnp.float32)
    lane = jax.lax.broadcasted_iota(jnp.int32, (1, _PAD), 1)

    def body(i, carry):
        row = sup_ref[pl.ds(i, 1), :]
        keep = keep_ref[:, :]
        onehot = jnp.where(lane == i, 1.0, 0.0)
        ki = jnp.sum(keep * onehot)
        keep_ref[:, :] = keep * (1.0 - row * ki)
        return carry

    jax.lax.fori_loop(0, _TOPK, body, 0)

    keep = keep_ref[:, :]
    sc = sc_ref[:, :]
    valid = (keep > 0.5) & (sc > _SCORE_THRESH)
    out_ref[:, :] = jnp.where(valid, sc, -jnp.inf)


def _run_nms(bc, br, sc):
    return pl.pallas_call(
        _nms_kernel,
        out_shape=jax.ShapeDtypeStruct((1, _PAD), jnp.float32),
        scratch_shapes=[
            pltpu.VMEM((_PAD, _PAD), jnp.float32),
            pltpu.VMEM((1, _PAD), jnp.float32),
        ],
    )(bc, br, sc)


def kernel(boxes, scores):
    masked = jnp.where(scores > _SCORE_THRESH, scores, -jnp.inf)
    top_scores, top_idx = jax.lax.top_k(masked, _TOPK)
    tb = boxes[top_idx]
    x1 = tb[:, 0] * 800.0
    y1 = tb[:, 1] * 800.0
    x2 = x1 + tb[:, 2] * 512.0 + 1.0
    y2 = y1 + tb[:, 3] * 512.0 + 1.0
    bxyxy = jnp.stack([x1, y1, x2, y2], axis=1)

    bc = jnp.pad(bxyxy, ((0, _PAD - _TOPK), (0, 0)))
    br = bc.T
    sc = jnp.pad(top_scores, (0, _PAD - _TOPK), constant_values=-jnp.inf)[None, :]

    final = _run_nms(bc, br, sc)[0, :_TOPK]
    det_scores, det_idx = jax.lax.top_k(final, _DET_PER_IMG)
    det_boxes = bxyxy[det_idx]
    det_scores = jnp.where(jnp.isfinite(det_scores), det_scores, 0.0)
    return jnp.concatenate([det_boxes, det_scores[:, None]], axis=1)
